# baseline (speedup 1.0000x reference)
import jax
import jax.numpy as jnp
from jax import lax
from jax.experimental import pallas as pl
from jax.experimental.pallas import tpu as pltpu

_B, _C, _R = 2, 96, 48
_S = 32 * 64 * 64
_CHA = 8192
_NCH = _S // _CHA
_CHB = 4096


def _mean_kernel(x_ref, part_ref):
    blk = x_ref[...]
    part_ref[...] = jnp.sum(blk, axis=1, keepdims=True).reshape(1, 1, 1, _C)


def _mlp_rank_onehot_kernel(part_ref, w1_ref, b1_ref, w2_ref, b2_ref, p_ref):
    part = part_ref[...].reshape(_B, _NCH, _C)
    y = jnp.sum(part, axis=1) * (1.0 / _S)
    z1 = lax.dot_general(y, w1_ref[...], (((1,), (1,)), ((), ())),
                         preferred_element_type=jnp.float32)
    z1 = z1 + b1_ref[...]
    z1 = jnp.where(z1 >= 0, z1, 0.01 * z1)
    z2 = lax.dot_general(z1, w2_ref[...], (((1,), (1,)), ((), ())),
                         preferred_element_type=jnp.float32)
    z2 = z2 + b2_ref[...]
    s = 1.0 / (1.0 + jnp.exp(-z2))

    for b in range(_B):
        vrow = s[b:b + 1, :]
        rmat = jnp.broadcast_to(vrow, (_C, _C))
        cmat = rmat.T
        irow = lax.broadcasted_iota(jnp.int32, (_C, _C), 0)
        jcol = lax.broadcasted_iota(jnp.int32, (_C, _C), 1)
        beforeT = (rmat > cmat) | ((rmat == cmat) & (jcol < irow))
        rank = jnp.sum(beforeT.astype(jnp.int32), axis=1,
                       keepdims=True)
        pio = lax.broadcasted_iota(jnp.int32, (_C, _R), 1)
        onehot = (jnp.broadcast_to(rank, (_C, _R)) == pio)
        p_ref[b] = onehot.astype(jnp.float32)


def _gather_mm_kernel(x_ref, p_ref, o_ref):
    x2 = x_ref[...].reshape(_CHB, _C)
    p2 = p_ref[...].reshape(_C, _R)
    o = lax.dot_general(p2, x2, (((0,), (1,)), ((), ())),
                        precision=lax.Precision.HIGHEST,
                        preferred_element_type=jnp.float32)
    o_ref[...] = o.reshape(1, _R, _CHB)


def kernel(x, w1, b1, w2, b2):
    b, c, d, h, w = x.shape
    xt = jnp.transpose(x, (0, 2, 3, 4, 1))
    xv = xt.reshape(_B, _S, _C)

    part = pl.pallas_call(
        _mean_kernel,
        grid=(_B, _NCH),
        in_specs=[pl.BlockSpec((1, _CHA, _C), lambda i, j: (i, j, 0))],
        out_specs=pl.BlockSpec((1, 1, 1, _C), lambda i, j: (i, j, 0, 0)),
        out_shape=jax.ShapeDtypeStruct((_B, _NCH, 1, _C), jnp.float32),
    )(xv)

    onehot = pl.pallas_call(
        _mlp_rank_onehot_kernel,
        in_specs=[
            pl.BlockSpec((_B, _NCH, 1, _C), lambda: (0, 0, 0, 0)),
            pl.BlockSpec((_C, _C), lambda: (0, 0)),
            pl.BlockSpec((1, _C), lambda: (0, 0)),
            pl.BlockSpec((_C, _C), lambda: (0, 0)),
            pl.BlockSpec((1, _C), lambda: (0, 0)),
        ],
        out_specs=pl.BlockSpec((_B, _C, _R), lambda: (0, 0, 0)),
        out_shape=jax.ShapeDtypeStruct((_B, _C, _R), jnp.float32),
    )(part, w1, b1.reshape(1, _C), w2, b2.reshape(1, _C))

    out_t = pl.pallas_call(
        _gather_mm_kernel,
        grid=(_B, _S // _CHB),
        in_specs=[
            pl.BlockSpec((1, _CHB, _C), lambda i, j: (i, j, 0)),
            pl.BlockSpec((1, _C, _R), lambda i, j: (i, 0, 0)),
        ],
        out_specs=pl.BlockSpec((1, _R, _CHB), lambda i, j: (i, 0, j)),
        out_shape=jax.ShapeDtypeStruct((_B, _R, _S), jnp.float32),
    )(xv, onehot)

    return out_t.reshape(b, _R, d, h, w)


# speedup vs baseline: 1.7836x; 1.7836x over previous
import jax
import jax.numpy as jnp
from jax import lax
from jax.experimental import pallas as pl
from jax.experimental.pallas import tpu as pltpu

_B, _C, _R = 2, 96, 48
_S = 32 * 64 * 64
_CHA = 8192
_NCH = _S // _CHA
_CHB = 4096


def _mean_kernel(x_ref, part_ref):
    blk = x_ref[...]
    part_ref[...] = jnp.sum(blk, axis=1, keepdims=True).reshape(1, 1, 1, _C)


def _mlp_rank_onehot_kernel(part_ref, w1_ref, b1_ref, w2_ref, b2_ref, p_ref):
    part = part_ref[...].reshape(_B, _NCH, _C)
    y = jnp.sum(part, axis=1) * (1.0 / _S)
    z1 = lax.dot_general(y, w1_ref[...], (((1,), (1,)), ((), ())),
                         preferred_element_type=jnp.float32)
    z1 = z1 + b1_ref[...]
    z1 = jnp.where(z1 >= 0, z1, 0.01 * z1)
    z2 = lax.dot_general(z1, w2_ref[...], (((1,), (1,)), ((), ())),
                         preferred_element_type=jnp.float32)
    z2 = z2 + b2_ref[...]
    s = 1.0 / (1.0 + jnp.exp(-z2))

    for b in range(_B):
        vrow = s[b:b + 1, :]
        rmat = jnp.broadcast_to(vrow, (_C, _C))
        cmat = rmat.T
        irow = lax.broadcasted_iota(jnp.int32, (_C, _C), 0)
        jcol = lax.broadcasted_iota(jnp.int32, (_C, _C), 1)
        beforeT = (rmat > cmat) | ((rmat == cmat) & (jcol < irow))
        rank = jnp.sum(beforeT.astype(jnp.int32), axis=1,
                       keepdims=True)
        pio = lax.broadcasted_iota(jnp.int32, (_C, _R), 1)
        onehot = (jnp.broadcast_to(rank, (_C, _R)) == pio)
        p_ref[b] = onehot.astype(jnp.float32)


def _gather_mm_kernel(x_ref, p_ref, o_ref):
    x3 = x_ref[...].reshape(64, 64, _C)
    p2 = p_ref[...].reshape(_C, _R)
    xh = x3.astype(jnp.bfloat16).astype(jnp.float32)
    xl = x3 - xh
    dn = (((0,), (2,)), ((), ()))
    o = (lax.dot_general(p2, xh, dn, preferred_element_type=jnp.float32)
         + lax.dot_general(p2, xl, dn, preferred_element_type=jnp.float32))
    o_ref[...] = o.reshape(1, _R, 1, 64, 64)


def kernel(x, w1, b1, w2, b2):
    b, c, d, h, w = x.shape
    xt = jnp.transpose(x, (0, 2, 3, 4, 1))
    xv = xt.reshape(_B, _S, _C)

    part = pl.pallas_call(
        _mean_kernel,
        grid=(_B, _NCH),
        in_specs=[pl.BlockSpec((1, _CHA, _C), lambda i, j: (i, j, 0))],
        out_specs=pl.BlockSpec((1, 1, 1, _C), lambda i, j: (i, j, 0, 0)),
        out_shape=jax.ShapeDtypeStruct((_B, _NCH, 1, _C), jnp.float32),
    )(xv)

    onehot = pl.pallas_call(
        _mlp_rank_onehot_kernel,
        in_specs=[
            pl.BlockSpec((_B, _NCH, 1, _C), lambda: (0, 0, 0, 0)),
            pl.BlockSpec((_C, _C), lambda: (0, 0)),
            pl.BlockSpec((1, _C), lambda: (0, 0)),
            pl.BlockSpec((_C, _C), lambda: (0, 0)),
            pl.BlockSpec((1, _C), lambda: (0, 0)),
        ],
        out_specs=pl.BlockSpec((_B, _C, _R), lambda: (0, 0, 0)),
        out_shape=jax.ShapeDtypeStruct((_B, _C, _R), jnp.float32),
    )(part, w1, b1.reshape(1, _C), w2, b2.reshape(1, _C))

    x5 = xt.reshape(_B, d, h, w, _C)
    out = pl.pallas_call(
        _gather_mm_kernel,
        grid=(_B, d),
        in_specs=[
            pl.BlockSpec((1, 1, h, w, _C), lambda i, j: (i, j, 0, 0, 0)),
            pl.BlockSpec((1, _C, _R), lambda i, j: (i, 0, 0)),
        ],
        out_specs=pl.BlockSpec((1, _R, 1, h, w), lambda i, j: (i, 0, j, 0, 0)),
        out_shape=jax.ShapeDtypeStruct((_B, _R, d, h, w), jnp.float32),
    )(x5, onehot)

    return out


# speedup vs baseline: 2.0718x; 1.1616x over previous
import jax
import jax.numpy as jnp
from jax import lax
from jax.experimental import pallas as pl
from jax.experimental.pallas import tpu as pltpu

_B, _C, _R = 2, 96, 48
_S = 32 * 64 * 64
_CHA = 16384
_NCH = _S // _CHA
_DCH = 2


def _mean_kernel(x_ref, part_ref):
    blk = x_ref[...]
    part_ref[...] = jnp.sum(blk, axis=1, keepdims=True).reshape(1, 1, 1, _C)


def _mlp_rank_onehot_kernel(part_ref, w1_ref, b1_ref, w2_ref, b2_ref, p_ref):
    part = part_ref[...].reshape(_B, _NCH, _C)
    y = jnp.sum(part, axis=1) * (1.0 / _S)
    z1 = lax.dot_general(y, w1_ref[...], (((1,), (1,)), ((), ())),
                         preferred_element_type=jnp.float32)
    z1 = z1 + b1_ref[...]
    z1 = jnp.where(z1 >= 0, z1, 0.01 * z1)
    z2 = lax.dot_general(z1, w2_ref[...], (((1,), (1,)), ((), ())),
                         preferred_element_type=jnp.float32)
    z2 = z2 + b2_ref[...]
    s = 1.0 / (1.0 + jnp.exp(-z2))

    for b in range(_B):
        vrow = s[b:b + 1, :]
        rmat = jnp.broadcast_to(vrow, (_C, _C))
        cmat = rmat.T
        irow = lax.broadcasted_iota(jnp.int32, (_C, _C), 0)
        jcol = lax.broadcasted_iota(jnp.int32, (_C, _C), 1)
        beforeT = (rmat > cmat) | ((rmat == cmat) & (jcol < irow))
        rank = jnp.sum(beforeT.astype(jnp.int32), axis=1,
                       keepdims=True)
        pio = lax.broadcasted_iota(jnp.int32, (_C, _R), 1)
        onehot = (jnp.broadcast_to(rank, (_C, _R)) == pio)
        p_ref[b] = onehot.astype(jnp.float32)


def _gather_mm_kernel(x_ref, p_ref, o_ref):
    x3 = x_ref[...].reshape(_DCH, 64, 64, _C)
    p2 = p_ref[...].reshape(_C, _R)
    xh = x3.astype(jnp.bfloat16).astype(jnp.float32)
    xl = x3 - xh
    dn = (((0,), (3,)), ((), ()))
    o = (lax.dot_general(p2, xh, dn, preferred_element_type=jnp.float32)
         + lax.dot_general(p2, xl, dn, preferred_element_type=jnp.float32))
    o_ref[...] = o.reshape(1, _R, _DCH, 64, 64)


def kernel(x, w1, b1, w2, b2):
    b, c, d, h, w = x.shape
    xt = jnp.transpose(x, (0, 2, 3, 4, 1))
    xv = xt.reshape(_B, _S, _C)

    part = pl.pallas_call(
        _mean_kernel,
        grid=(_B, _NCH),
        in_specs=[pl.BlockSpec((1, _CHA, _C), lambda i, j: (i, j, 0))],
        out_specs=pl.BlockSpec((1, 1, 1, _C), lambda i, j: (i, j, 0, 0)),
        out_shape=jax.ShapeDtypeStruct((_B, _NCH, 1, _C), jnp.float32),
    )(xv)

    onehot = pl.pallas_call(
        _mlp_rank_onehot_kernel,
        in_specs=[
            pl.BlockSpec((_B, _NCH, 1, _C), lambda: (0, 0, 0, 0)),
            pl.BlockSpec((_C, _C), lambda: (0, 0)),
            pl.BlockSpec((1, _C), lambda: (0, 0)),
            pl.BlockSpec((_C, _C), lambda: (0, 0)),
            pl.BlockSpec((1, _C), lambda: (0, 0)),
        ],
        out_specs=pl.BlockSpec((_B, _C, _R), lambda: (0, 0, 0)),
        out_shape=jax.ShapeDtypeStruct((_B, _C, _R), jnp.float32),
    )(part, w1, b1.reshape(1, _C), w2, b2.reshape(1, _C))

    x5 = xt.reshape(_B, d, h, w, _C)
    out = pl.pallas_call(
        _gather_mm_kernel,
        grid=(_B, d // _DCH),
        in_specs=[
            pl.BlockSpec((1, _DCH, h, w, _C), lambda i, j: (i, j, 0, 0, 0)),
            pl.BlockSpec((1, _C, _R), lambda i, j: (i, 0, 0)),
        ],
        out_specs=pl.BlockSpec((1, _R, _DCH, h, w),
                               lambda i, j: (i, 0, j, 0, 0)),
        out_shape=jax.ShapeDtypeStruct((_B, _R, d, h, w), jnp.float32),
    )(x5, onehot)

    return out


# speedup vs baseline: 2.2083x; 1.0659x over previous
import jax
import jax.numpy as jnp
from jax import lax
from jax.experimental import pallas as pl
from jax.experimental.pallas import tpu as pltpu

_B, _C, _R = 2, 96, 48
_S = 32 * 64 * 64
_CHA = 32768
_NCH = _S // _CHA
_DCH = 4


def _mean_kernel(x_ref, part_ref):
    blk = x_ref[...]
    part_ref[...] = jnp.sum(blk, axis=1, keepdims=True).reshape(1, 1, 1, _C)


def _mlp_rank_onehot_kernel(part_ref, w1_ref, b1_ref, w2_ref, b2_ref, p_ref):
    part = part_ref[...].reshape(_B, _NCH, _C)
    y = jnp.sum(part, axis=1) * (1.0 / _S)
    z1 = lax.dot_general(y, w1_ref[...], (((1,), (1,)), ((), ())),
                         preferred_element_type=jnp.float32)
    z1 = z1 + b1_ref[...]
    z1 = jnp.where(z1 >= 0, z1, 0.01 * z1)
    z2 = lax.dot_general(z1, w2_ref[...], (((1,), (1,)), ((), ())),
                         preferred_element_type=jnp.float32)
    z2 = z2 + b2_ref[...]
    s = 1.0 / (1.0 + jnp.exp(-z2))

    for b in range(_B):
        vrow = s[b:b + 1, :]
        rmat = jnp.broadcast_to(vrow, (_C, _C))
        cmat = rmat.T
        irow = lax.broadcasted_iota(jnp.int32, (_C, _C), 0)
        jcol = lax.broadcasted_iota(jnp.int32, (_C, _C), 1)
        beforeT = (rmat > cmat) | ((rmat == cmat) & (jcol < irow))
        rank = jnp.sum(beforeT.astype(jnp.int32), axis=1,
                       keepdims=True)
        pio = lax.broadcasted_iota(jnp.int32, (_C, _R), 1)
        onehot = (jnp.broadcast_to(rank, (_C, _R)) == pio)
        p_ref[b] = onehot.astype(jnp.float32)


def _gather_mm_kernel(x_ref, p_ref, o_ref):
    x3 = x_ref[...].reshape(_DCH, 64, 64, _C)
    p2 = p_ref[...].reshape(_C, _R)
    xh = x3.astype(jnp.bfloat16).astype(jnp.float32)
    xl = x3 - xh
    dn = (((0,), (3,)), ((), ()))
    o = (lax.dot_general(p2, xh, dn, preferred_element_type=jnp.float32)
         + lax.dot_general(p2, xl, dn, preferred_element_type=jnp.float32))
    o_ref[...] = o.reshape(1, _R, _DCH, 64, 64)


def kernel(x, w1, b1, w2, b2):
    b, c, d, h, w = x.shape
    xt = jnp.transpose(x, (0, 2, 3, 4, 1))
    xv = xt.reshape(_B, _S, _C)

    part = pl.pallas_call(
        _mean_kernel,
        grid=(_B, _NCH),
        in_specs=[pl.BlockSpec((1, _CHA, _C), lambda i, j: (i, j, 0))],
        out_specs=pl.BlockSpec((1, 1, 1, _C), lambda i, j: (i, j, 0, 0)),
        out_shape=jax.ShapeDtypeStruct((_B, _NCH, 1, _C), jnp.float32),
    )(xv)

    onehot = pl.pallas_call(
        _mlp_rank_onehot_kernel,
        in_specs=[
            pl.BlockSpec((_B, _NCH, 1, _C), lambda: (0, 0, 0, 0)),
            pl.BlockSpec((_C, _C), lambda: (0, 0)),
            pl.BlockSpec((1, _C), lambda: (0, 0)),
            pl.BlockSpec((_C, _C), lambda: (0, 0)),
            pl.BlockSpec((1, _C), lambda: (0, 0)),
        ],
        out_specs=pl.BlockSpec((_B, _C, _R), lambda: (0, 0, 0)),
        out_shape=jax.ShapeDtypeStruct((_B, _C, _R), jnp.float32),
    )(part, w1, b1.reshape(1, _C), w2, b2.reshape(1, _C))

    x5 = xt.reshape(_B, d, h, w, _C)
    out = pl.pallas_call(
        _gather_mm_kernel,
        grid=(_B, d // _DCH),
        in_specs=[
            pl.BlockSpec((1, _DCH, h, w, _C), lambda i, j: (i, j, 0, 0, 0)),
            pl.BlockSpec((1, _C, _R), lambda i, j: (i, 0, 0)),
        ],
        out_specs=pl.BlockSpec((1, _R, _DCH, h, w),
                               lambda i, j: (i, 0, j, 0, 0)),
        out_shape=jax.ShapeDtypeStruct((_B, _R, d, h, w), jnp.float32),
    )(x5, onehot)

    return out


# speedup vs baseline: 2.2863x; 1.0353x over previous
import jax
import jax.numpy as jnp
from jax import lax
from jax.experimental import pallas as pl
from jax.experimental.pallas import tpu as pltpu

_B, _C, _R = 2, 96, 48
_S = 32 * 64 * 64
_CHA = 16384
_NCH = _S // _CHA
_DCH = 4


def _mean_mlp_onehot_kernel(x_ref, w1_ref, b1_ref, w2_ref, b2_ref,
                            p_ref, acc_ref):
    j = pl.program_id(0)
    blk = x_ref[...]
    sums = jnp.sum(blk, axis=1)

    @pl.when(j == 0)
    def _init():
        acc_ref[...] = sums

    @pl.when(j > 0)
    def _acc():
        acc_ref[...] = acc_ref[...] + sums

    @pl.when(j == _NCH - 1)
    def _finish():
        y = acc_ref[...] * (1.0 / _S)
        z1 = lax.dot_general(y, w1_ref[...], (((1,), (1,)), ((), ())),
                             preferred_element_type=jnp.float32)
        z1 = z1 + b1_ref[...]
        z1 = jnp.where(z1 >= 0, z1, 0.01 * z1)
        z2 = lax.dot_general(z1, w2_ref[...], (((1,), (1,)), ((), ())),
                             preferred_element_type=jnp.float32)
        z2 = z2 + b2_ref[...]
        s = 1.0 / (1.0 + jnp.exp(-z2))

        for b in range(_B):
            vrow = s[b:b + 1, :]
            rmat = jnp.broadcast_to(vrow, (_C, _C))
            cmat = rmat.T
            irow = lax.broadcasted_iota(jnp.int32, (_C, _C), 0)
            jcol = lax.broadcasted_iota(jnp.int32, (_C, _C), 1)
            beforeT = (rmat > cmat) | ((rmat == cmat) & (jcol < irow))
            rank = jnp.sum(beforeT.astype(jnp.int32), axis=1,
                           keepdims=True)
            pio = lax.broadcasted_iota(jnp.int32, (_C, _R), 1)
            onehot = (jnp.broadcast_to(rank, (_C, _R)) == pio)
            p_ref[b] = onehot.astype(jnp.float32)


def _gather_mm_kernel(x_ref, p_ref, o_ref):
    x3 = x_ref[...].reshape(_DCH, 64, 64, _C)
    p2 = p_ref[...].reshape(_C, _R)
    xh = x3.astype(jnp.bfloat16).astype(jnp.float32)
    xl = x3 - xh
    dn = (((0,), (3,)), ((), ()))
    o = (lax.dot_general(p2, xh, dn, preferred_element_type=jnp.float32)
         + lax.dot_general(p2, xl, dn, preferred_element_type=jnp.float32))
    o_ref[...] = o.reshape(1, _R, _DCH, 64, 64)


def kernel(x, w1, b1, w2, b2):
    b, c, d, h, w = x.shape
    xt = jnp.transpose(x, (0, 2, 3, 4, 1))
    xv = xt.reshape(_B, _S, _C)

    onehot = pl.pallas_call(
        _mean_mlp_onehot_kernel,
        grid=(_NCH,),
        in_specs=[
            pl.BlockSpec((_B, _CHA, _C), lambda j: (0, j, 0)),
            pl.BlockSpec((_C, _C), lambda j: (0, 0)),
            pl.BlockSpec((1, _C), lambda j: (0, 0)),
            pl.BlockSpec((_C, _C), lambda j: (0, 0)),
            pl.BlockSpec((1, _C), lambda j: (0, 0)),
        ],
        out_specs=pl.BlockSpec((_B, _C, _R), lambda j: (0, 0, 0)),
        out_shape=jax.ShapeDtypeStruct((_B, _C, _R), jnp.float32),
        scratch_shapes=[pltpu.VMEM((_B, _C), jnp.float32)],
    )(xv, w1, b1.reshape(1, _C), w2, b2.reshape(1, _C))

    x5 = xt.reshape(_B, d, h, w, _C)
    out = pl.pallas_call(
        _gather_mm_kernel,
        grid=(_B, d // _DCH),
        in_specs=[
            pl.BlockSpec((1, _DCH, h, w, _C), lambda i, j: (i, j, 0, 0, 0)),
            pl.BlockSpec((1, _C, _R), lambda i, j: (i, 0, 0)),
        ],
        out_specs=pl.BlockSpec((1, _R, _DCH, h, w),
                               lambda i, j: (i, 0, j, 0, 0)),
        out_shape=jax.ShapeDtypeStruct((_B, _R, d, h, w), jnp.float32),
    )(x5, onehot)

    return out
